# baseline (speedup 1.0000x reference)
import functools

import jax
import jax.numpy as jnp
from jax import lax
from jax.experimental import pallas as pl
from jax.experimental.pallas import tpu as pltpu
from jax.experimental.pallas import tpu_sc as plsc

_NC = 2
_NS = 16
_NW = _NC * _NS
_C = 128


@functools.partial(jax.jit, static_argnums=(2, 3))
def _sc_gather(ids3, table, n_per_w, n_chunks):
    d = table.shape[1]
    n = n_per_w * _NW
    mesh = plsc.VectorSubcoreMesh(core_axis_name="c", subcore_axis_name="s")

    @functools.partial(
        pl.kernel,
        mesh=mesh,
        out_type=jax.ShapeDtypeStruct((n, d), jnp.float32),
        scratch_types=[
            pltpu.VMEM((n_chunks, _C), jnp.int32),
            pltpu.VMEM((_C, d), jnp.float32),
            pltpu.SemaphoreType.DMA,
        ],
        compiler_params=pltpu.CompilerParams(use_tc_tiling_on_sc=False),
    )
    def k(ids_hbm, table_hbm, out_hbm, idx_v, rows_v, sem):
        wid = lax.axis_index("s") * _NC + lax.axis_index("c")
        base = wid * n_per_w
        pltpu.sync_copy(ids_hbm.at[wid], idx_v)

        def body(j, carry):
            pltpu.async_copy(table_hbm.at[idx_v.at[j]], rows_v, sem).wait()
            pltpu.sync_copy(rows_v, out_hbm.at[pl.ds(base + j * _C, _C)])
            return carry

        lax.fori_loop(0, n_chunks, body, 0)

    return k(ids3, table)


def kernel(input_ids, table):
    b, l = input_ids.shape
    n = b * l
    assert n % (_NW * _C) == 0
    n_per_w = n // _NW
    n_chunks = n_per_w // _C
    ids3 = input_ids.reshape(_NW, n_chunks, _C)
    out = _sc_gather(ids3, table, n_per_w, n_chunks)
    return out.reshape(b, l, table.shape[1])


# speedup vs baseline: 1.1188x; 1.1188x over previous
import functools

import jax
import jax.numpy as jnp
from jax import lax
from jax.experimental import pallas as pl
from jax.experimental.pallas import tpu as pltpu
from jax.experimental.pallas import tpu_sc as plsc

_NC = 2
_NS = 16
_NW = _NC * _NS
_C = 128
_NBUF = 8


@functools.partial(jax.jit, static_argnums=(2, 3))
def _sc_gather(ids3, table, n_per_w, n_chunks):
    d = table.shape[1]
    n = n_per_w * _NW
    n_groups = n_chunks // _NBUF
    mesh = plsc.VectorSubcoreMesh(core_axis_name="c", subcore_axis_name="s")

    scratch = (
        [pltpu.VMEM((n_chunks, _C), jnp.int32)]
        + [pltpu.VMEM((_C, d), jnp.float32) for _ in range(_NBUF)]
        + [pltpu.SemaphoreType.DMA for _ in range(2 * _NBUF)]
    )

    @functools.partial(
        pl.kernel,
        mesh=mesh,
        out_type=jax.ShapeDtypeStruct((n, d), jnp.float32),
        scratch_types=scratch,
        compiler_params=pltpu.CompilerParams(use_tc_tiling_on_sc=False),
    )
    def k(ids_hbm, table_hbm, out_hbm, idx_v, *rest):
        bufs = rest[:_NBUF]
        in_sem = rest[_NBUF:2 * _NBUF]
        out_sem = rest[2 * _NBUF:]
        wid = lax.axis_index("s") * _NC + lax.axis_index("c")
        base = wid * n_per_w
        pltpu.sync_copy(ids_hbm.at[wid], idx_v)

        def start_gather(jn, b):
            pltpu.async_copy(table_hbm.at[idx_v.at[jn]], bufs[b], in_sem[b])

        def wait_gather(b):
            pltpu.make_async_copy(
                table_hbm.at[idx_v.at[0]], bufs[b], in_sem[b]).wait()

        def start_scatter(j, b):
            pltpu.async_copy(
                bufs[b], out_hbm.at[pl.ds(base + j * _C, _C)], out_sem[b])

        def wait_scatter(b):
            pltpu.make_async_copy(
                bufs[b], out_hbm.at[pl.ds(base, _C)], out_sem[b]).wait()

        for b in range(_NBUF - 1):
            start_gather(b, b)
        start_scatter(_NBUF - 1, _NBUF - 1)

        def visit(j, b):
            b1 = (b - 1) % _NBUF
            wait_scatter(b1)
            start_gather(j + _NBUF - 1, b1)
            wait_gather(b)
            start_scatter(j, b)

        def body(g, carry):
            for b in range(_NBUF):
                visit(g * _NBUF + b, b)
            return carry

        lax.fori_loop(0, n_groups - 1, body, 0)

        g0 = (n_groups - 1) * _NBUF
        for b in range(_NBUF):
            b1 = (b - 1) % _NBUF
            if b == 0:
                wait_scatter(b1)
                start_gather(g0 + _NBUF - 1, b1)
            wait_gather(b)
            start_scatter(g0 + b, b)
        for b in range(_NBUF):
            wait_scatter(b)

    return k(ids3, table)


def kernel(input_ids, table):
    b, l = input_ids.shape
    n = b * l
    assert n % (_NW * _C * _NBUF) == 0
    n_per_w = n // _NW
    n_chunks = n_per_w // _C
    ids3 = input_ids.reshape(_NW, n_chunks, _C)
    out = _sc_gather(ids3, table, n_per_w, n_chunks)
    return out.reshape(b, l, table.shape[1])


# speedup vs baseline: 1.3646x; 1.2197x over previous
import functools

import jax
import jax.numpy as jnp
from jax import lax
from jax.experimental import pallas as pl
from jax.experimental.pallas import tpu as pltpu
from jax.experimental.pallas import tpu_sc as plsc

_NC = 2
_NS = 16
_NW = _NC * _NS
_C = 128
_NBUF = 4


@functools.partial(jax.jit, static_argnums=(2, 3))
def _sc_gather(ids3, tableP, n_per_w, n_chunks):
    d = tableP.shape[1]
    n = n_per_w * _NW
    n_groups = n_chunks // _NBUF
    mesh = plsc.VectorSubcoreMesh(core_axis_name="c", subcore_axis_name="s")

    scratch = (
        [pltpu.VMEM((n_chunks, _C), jnp.int32)]
        + [pltpu.VMEM((_C, d), jnp.float32) for _ in range(_NBUF)]
        + [pltpu.SemaphoreType.DMA for _ in range(2 * _NBUF)]
    )

    @functools.partial(
        pl.kernel,
        mesh=mesh,
        out_type=jax.ShapeDtypeStruct((n, d), jnp.float32),
        scratch_types=scratch,
        compiler_params=pltpu.CompilerParams(use_tc_tiling_on_sc=True),
    )
    def k(ids_hbm, table_hbm, out_hbm, idx_v, *rest):
        bufs = rest[:_NBUF]
        in_sem = rest[_NBUF:2 * _NBUF]
        out_sem = rest[2 * _NBUF:]
        wid = lax.axis_index("s") * _NC + lax.axis_index("c")
        base = wid * n_per_w
        pltpu.sync_copy(ids_hbm.at[wid], idx_v)

        def start_gather(jn, b):
            pltpu.async_copy(table_hbm.at[idx_v.at[jn]], bufs[b], in_sem[b])

        def wait_gather(b):
            pltpu.make_async_copy(
                table_hbm.at[idx_v.at[0]], bufs[b], in_sem[b]).wait()

        def start_scatter(j, b):
            pltpu.async_copy(
                bufs[b], out_hbm.at[pl.ds(base + j * _C, _C)], out_sem[b])

        def wait_scatter(b):
            pltpu.make_async_copy(
                bufs[b], out_hbm.at[pl.ds(base, _C)], out_sem[b]).wait()

        for b in range(_NBUF - 1):
            start_gather(b, b)
        start_scatter(_NBUF - 1, _NBUF - 1)

        def visit(j, b):
            b1 = (b - 1) % _NBUF
            wait_scatter(b1)
            start_gather(j + _NBUF - 1, b1)
            wait_gather(b)
            start_scatter(j, b)

        def body(g, carry):
            for b in range(_NBUF):
                visit(g * _NBUF + b, b)
            return carry

        lax.fori_loop(0, n_groups - 1, body, 0)

        g0 = (n_groups - 1) * _NBUF
        for b in range(_NBUF):
            b1 = (b - 1) % _NBUF
            if b == 0:
                wait_scatter(b1)
                start_gather(g0 + _NBUF - 1, b1)
            wait_gather(b)
            start_scatter(g0 + b, b)
        for b in range(_NBUF):
            wait_scatter(b)

    return k(ids3, tableP)


def kernel(input_ids, table):
    b, l = input_ids.shape
    n = b * l
    assert n % (_NW * _C * _NBUF) == 0
    n_per_w = n // _NW
    n_chunks = n_per_w // _C
    ids3 = input_ids.reshape(_NW, n_chunks, _C)
    d = table.shape[1]
    tableP = jnp.pad(table, ((0, 0), (0, 128 - d)))
    outP = _sc_gather(ids3, tableP, n_per_w, n_chunks)
    out = lax.slice(outP, (0, 0), (n, d))
    return out.reshape(b, l, d)
